# baseline (speedup 1.0000x reference)
import functools

import jax
import jax.numpy as jnp
from jax import lax
from jax.experimental import pallas as pl
from jax.experimental.pallas import tpu as pltpu
from jax.experimental.pallas import tpu_sc as plsc

NW = 32
CHUNK = 128
_SC_PARAMS = pltpu.CompilerParams(needs_layout_passes=False)


def _matmul(x, w, mb):
    m, k = x.shape
    _, n = w.shape

    def body(x_ref, w_ref, o_ref):
        o_ref[...] = jnp.dot(x_ref[...], w_ref[...],
                             preferred_element_type=jnp.float32)

    return pl.pallas_call(
        body,
        grid=(m // mb,),
        in_specs=[pl.BlockSpec((mb, k), lambda i: (i, 0)),
                  pl.BlockSpec((k, n), lambda i: (0, 0))],
        out_specs=pl.BlockSpec((mb, n), lambda i: (i, 0)),
        out_shape=jax.ShapeDtypeStruct((m, n), jnp.float32),
    )(x, w)


def _matmul_scale(x, w, degp_t, mb):
    m, k = x.shape
    _, n = w.shape
    p = degp_t.shape[1]

    def body(x_ref, w_ref, d_ref, y_ref, dinv_ref):
        deg = jnp.sum(d_ref[...], axis=1) + 1.0
        dinv = lax.rsqrt(deg)
        acc = jnp.dot(x_ref[...], w_ref[...], preferred_element_type=jnp.float32)
        y_ref[...] = acc * dinv[:, None]
        dinv_ref[...] = dinv[:, None]

    return pl.pallas_call(
        body,
        grid=(m // mb,),
        in_specs=[pl.BlockSpec((mb, k), lambda i: (i, 0)),
                  pl.BlockSpec((k, n), lambda i: (0, 0)),
                  pl.BlockSpec((mb, p), lambda i: (i, 0))],
        out_specs=[pl.BlockSpec((mb, n), lambda i: (i, 0)),
                   pl.BlockSpec((mb, 1), lambda i: (i, 0))],
        out_shape=[jax.ShapeDtypeStruct((m, n), jnp.float32),
                   jax.ShapeDtypeStruct((m, 1), jnp.float32)],
    )(x, w, degp_t)


def _layer_mm(sp, y, dinv, b, w):
    _, n, h = sp.shape
    mb = 1000

    def body(sp_ref, y_ref, di_ref, b_ref, w_ref, o_ref):
        t = di_ref[...] * (sp_ref[0] + sp_ref[1] + y_ref[...]) + b_ref[...]
        hh = jnp.maximum(t, 0.0)
        o_ref[...] = jnp.dot(hh, w_ref[...],
                             preferred_element_type=jnp.float32) * di_ref[...]

    return pl.pallas_call(
        body,
        grid=(n // mb,),
        in_specs=[pl.BlockSpec((2, mb, h), lambda i: (0, i, 0)),
                  pl.BlockSpec((mb, h), lambda i: (i, 0)),
                  pl.BlockSpec((mb, 1), lambda i: (i, 0)),
                  pl.BlockSpec((h,), lambda i: (0,)),
                  pl.BlockSpec((h, h), lambda i: (0, 0))],
        out_specs=pl.BlockSpec((mb, h), lambda i: (i, 0)),
        out_shape=jax.ShapeDtypeStruct((n, h), jnp.float32),
    )(sp, y, dinv, b, w)


def _layer_mm_final(sp, y, dinv, b, wcat, bl1):
    _, n, h = sp.shape
    mb = 1000

    def body(sp_ref, y_ref, di_ref, b_ref, w_ref, bl1_ref, t_ref):
        t = di_ref[...] * (sp_ref[0] + sp_ref[1] + y_ref[...]) + b_ref[...]
        hh = jnp.maximum(t, 0.0)
        acc = jnp.dot(hh, w_ref[...], preferred_element_type=jnp.float32)
        t_ref[...] = (acc + jnp.concatenate(
            [bl1_ref[...], jnp.zeros_like(bl1_ref[...])])).astype(jnp.bfloat16)

    return pl.pallas_call(
        body,
        grid=(n // mb,),
        in_specs=[pl.BlockSpec((2, mb, h), lambda i: (0, i, 0)),
                  pl.BlockSpec((mb, h), lambda i: (i, 0)),
                  pl.BlockSpec((mb, 1), lambda i: (i, 0)),
                  pl.BlockSpec((h,), lambda i: (0,)),
                  pl.BlockSpec((h, 2 * h), lambda i: (0, 0)),
                  pl.BlockSpec((h,), lambda i: (0,))],
        out_specs=pl.BlockSpec((mb, 2 * h), lambda i: (i, 0)),
        out_shape=jax.ShapeDtypeStruct((n, 2 * h), jnp.bfloat16),
    )(sp, y, dinv, b, wcat, bl1)


def _final(z, wl2, bl2):
    e, h = z.shape
    c = wl2.shape[1]
    mb = 4000

    def body(z_ref, w_ref, b_ref, o_ref):
        zz = jnp.maximum(z_ref[...], 0.0)
        l = jnp.dot(zz, w_ref[...], preferred_element_type=jnp.float32) + b_ref[...]
        m = jnp.max(l, axis=1, keepdims=True)
        s = l - m
        lse = jnp.log(jnp.sum(jnp.exp(s), axis=1, keepdims=True))
        o_ref[...] = s - lse

    return pl.pallas_call(
        body,
        grid=(e // mb,),
        in_specs=[pl.BlockSpec((mb, h), lambda i: (i, 0)),
                  pl.BlockSpec((h, c), lambda i: (0, 0)),
                  pl.BlockSpec((c,), lambda i: (0,))],
        out_specs=pl.BlockSpec((mb, c), lambda i: (i, 0)),
        out_shape=jax.ShapeDtypeStruct((e, c), jnp.float32),
    )(z, wl2, bl2)


def _sc_degree(dst, n):
    e = dst.shape[0]
    epw = e // NW
    full = epw // 16
    rem = epw - full * 16
    mesh = plsc.VectorSubcoreMesh(core_axis_name="c", subcore_axis_name="s")

    @functools.partial(
        pl.kernel,
        out_type=jax.ShapeDtypeStruct((NW, n), jnp.float32),
        mesh=mesh,
        compiler_params=_SC_PARAMS,
        scratch_types=[
            pltpu.VMEM((epw + 16,), jnp.int32),
            pltpu.VMEM((n,), jnp.float32),
        ],
    )
    def k(dst_hbm, out_hbm, idx_v, hist_v):
        cid = lax.axis_index("c")
        sid = lax.axis_index("s")
        wid = sid * 2 + cid
        zeros16 = jnp.zeros((16,), jnp.float32)
        ones16 = jnp.ones((16,), jnp.float32)

        def zero_body(i, _):
            hist_v[pl.ds(i * 16, 16)] = zeros16
            return 0
        lax.fori_loop(0, n // 16, zero_body, 0)

        pltpu.sync_copy(dst_hbm.at[pl.ds(wid * epw, epw)], idx_v.at[pl.ds(0, epw)])

        def body(i, _):
            v = idx_v[pl.ds(i * 16, 16)]
            plsc.addupdate_scatter(hist_v, [v], ones16)
            return 0
        lax.fori_loop(0, full, body, 0)
        if rem:
            v = idx_v[pl.ds(full * 16, 16)]
            mask = lax.iota(jnp.int32, 16) < rem
            plsc.addupdate_scatter(hist_v, [v], ones16, mask=mask)

        pltpu.sync_copy(hist_v, out_hbm.at[wid])

    return k(dst)


def _sc_aggregate(y, src, dst, zeros):
    n, h = y.shape
    e = src.shape[0]
    epw = e // NW
    nfull = epw // CHUNK
    tail = epw - nfull * CHUNK
    rps = (n // (16 * 8)) * 8
    rextra = n - 16 * rps
    mesh = plsc.VectorSubcoreMesh(core_axis_name="c", subcore_axis_name="s")

    npairs = nfull // 2
    leftover = nfull - 2 * npairs

    @functools.partial(
        pl.kernel,
        out_type=jax.ShapeDtypeStruct((2, n, h), jnp.float32),
        mesh=mesh,
        compiler_params=_SC_PARAMS,
        scratch_types=[
            pltpu.VMEM((epw + 16,), jnp.int32),
            pltpu.VMEM((epw + 16,), jnp.int32),
            pltpu.VMEM((CHUNK,), jnp.int32),
            pltpu.VMEM((CHUNK,), jnp.int32),
            pltpu.VMEM((CHUNK,), jnp.int32),
            pltpu.VMEM((CHUNK,), jnp.int32),
            pltpu.VMEM((CHUNK, h), jnp.float32),
            pltpu.VMEM((CHUNK, h), jnp.float32),
            pltpu.VMEM((max(tail, 1),), jnp.int32),
            pltpu.VMEM((max(tail, 1),), jnp.int32),
            pltpu.VMEM((max(tail, 1), h), jnp.float32),
            pltpu.VMEM_SHARED((n, h), jnp.float32),
            pltpu.SemaphoreType.DMA,
            pltpu.SemaphoreType.DMA,
            pltpu.SemaphoreType.DMA,
            pltpu.SemaphoreType.DMA,
        ],
    )
    def k(y_hbm, src_hbm, dst_hbm, zero_hbm, out_hbm,
          sall, dall, si0, si1, di0, di1, rows0, rows1,
          sidx_t, didx_t, rows_t, stab,
          semg0, semg1, sems0, sems1):
        cid = lax.axis_index("c")
        sid = lax.axis_index("s")
        wid = sid * 2 + cid
        base = wid * epw
        r0 = pl.multiple_of(sid * rps, 8)
        pltpu.sync_copy(zero_hbm.at[pl.ds(r0, rps)], stab.at[pl.ds(r0, rps)])
        if rextra:
            @pl.when(sid == 15)
            def _():
                pltpu.sync_copy(zero_hbm.at[pl.ds(16 * rps, rextra)],
                                stab.at[pl.ds(16 * rps, rextra)])
        pltpu.sync_copy(src_hbm.at[pl.ds(base, epw)], sall.at[pl.ds(0, epw)])
        pltpu.sync_copy(dst_hbm.at[pl.ds(base, epw)], dall.at[pl.ds(0, epw)])
        plsc.subcore_barrier()

        def stage(j, buf_all, buf_idx, m):
            for c in range(m // 16):
                buf_idx[pl.ds(c * 16, 16)] = buf_all[pl.ds(j * CHUNK + c * 16, 16)]

        def pair(t, _):
            a = 2 * t
            b = a + 1
            stage(a, sall, si0, CHUNK)
            stage(a, dall, di0, CHUNK)
            stage(b, sall, si1, CHUNK)
            stage(b, dall, di1, CHUNK)
            ga = pltpu.async_copy(y_hbm.at[si0], rows0, semg0)
            gb = pltpu.async_copy(y_hbm.at[si1], rows1, semg1)
            ga.wait()
            sa = pltpu.async_copy(rows0, stab.at[di0], sems0, add=True)
            gb.wait()
            sb = pltpu.async_copy(rows1, stab.at[di1], sems1, add=True)
            sa.wait()
            sb.wait()
            return 0
        lax.fori_loop(0, npairs, pair, 0)

        if leftover:
            j = 2 * npairs
            stage(j, sall, si0, CHUNK)
            stage(j, dall, di0, CHUNK)
            pltpu.async_copy(y_hbm.at[si0], rows0, semg0).wait()
            pltpu.async_copy(rows0, stab.at[di0], sems0, add=True).wait()
        if tail:
            off = base + nfull * CHUNK
            pltpu.sync_copy(src_hbm.at[pl.ds(off, tail)], sidx_t)
            pltpu.sync_copy(dst_hbm.at[pl.ds(off, tail)], didx_t)
            pltpu.async_copy(y_hbm.at[sidx_t], rows_t, semg1).wait()
            pltpu.async_copy(rows_t, stab.at[didx_t], sems1, add=True).wait()

        plsc.subcore_barrier()
        pltpu.sync_copy(stab.at[pl.ds(r0, rps)],
                        out_hbm.at[cid, pl.ds(r0, rps)])
        if rextra:
            @pl.when(sid == 15)
            def _():
                pltpu.sync_copy(stab.at[pl.ds(16 * rps, rextra)],
                                out_hbm.at[cid, pl.ds(16 * rps, rextra)])

    return k(y, src, dst, zeros)


def _sc_edge(t_packed, src, dst):
    n, h2 = t_packed.shape
    hw = h2 // 2
    zw = hw // 2
    e = src.shape[0]
    epw = e // NW
    nfull = epw // CHUNK
    tail = epw - nfull * CHUNK
    mesh = plsc.VectorSubcoreMesh(core_axis_name="c", subcore_axis_name="s")

    npairs = nfull // 2
    leftover = nfull - 2 * npairs

    @functools.partial(
        pl.kernel,
        out_type=jax.ShapeDtypeStruct((NW, epw, zw), jnp.int32),
        mesh=mesh,
        compiler_params=_SC_PARAMS,
        scratch_types=[
            pltpu.VMEM((epw + 16,), jnp.int32),
            pltpu.VMEM((epw + 16,), jnp.int32),
            pltpu.VMEM((CHUNK,), jnp.int32),
            pltpu.VMEM((CHUNK,), jnp.int32),
            pltpu.VMEM((CHUNK,), jnp.int32),
            pltpu.VMEM((CHUNK,), jnp.int32),
            pltpu.VMEM((CHUNK, hw), jnp.int32),
            pltpu.VMEM((CHUNK, hw), jnp.int32),
            pltpu.VMEM((CHUNK, hw), jnp.int32),
            pltpu.VMEM((CHUNK, hw), jnp.int32),
            pltpu.VMEM((CHUNK, zw), jnp.int32),
            pltpu.VMEM((CHUNK, zw), jnp.int32),
            pltpu.VMEM((max(tail, 1),), jnp.int32),
            pltpu.VMEM((max(tail, 1),), jnp.int32),
            pltpu.VMEM((max(tail, 1), hw), jnp.int32),
            pltpu.VMEM((max(tail, 1), hw), jnp.int32),
            pltpu.VMEM((max(tail, 1), zw), jnp.int32),
            pltpu.SemaphoreType.DMA,
            pltpu.SemaphoreType.DMA,
            pltpu.SemaphoreType.DMA,
            pltpu.SemaphoreType.DMA,
            pltpu.SemaphoreType.DMA,
            pltpu.SemaphoreType.DMA,
        ],
    )
    def k(t_hbm, src_hbm, dst_hbm, z_hbm,
          sall, dall, si0, di0, si1, di1, ra0, rb0, ra1, rb1, zb0, zb1,
          sidx_t, didx_t, ra_t, rb_t, zb_t,
          sga0, sgb0, sga1, sgb1, sw0, sw1):
        cid = lax.axis_index("c")
        sid = lax.axis_index("s")
        wid = sid * 2 + cid
        base = wid * epw
        pltpu.sync_copy(src_hbm.at[pl.ds(base, epw)], sall.at[pl.ds(0, epw)])
        pltpu.sync_copy(dst_hbm.at[pl.ds(base, epw)], dall.at[pl.ds(0, epw)])

        def stage(j, buf_all, buf_idx):
            for c in range(CHUNK // 16):
                buf_idx[pl.ds(c * 16, 16)] = buf_all[pl.ds(j * CHUNK + c * 16, 16)]

        def addrows(va, vb, zb, m):
            def addrow(r, _):
                for c in range(zw // 16):
                    xa = plsc.bitcast(va[r, pl.ds(c * 16, 16)], jnp.bfloat16)
                    xb = plsc.bitcast(vb[r, pl.ds(zw + c * 16, 16)], jnp.bfloat16)
                    zb[r, pl.ds(c * 16, 16)] = plsc.bitcast(xa + xb, jnp.int32)
                return 0
            lax.fori_loop(0, m, addrow, 0)

        def pair(t, _):
            a = 2 * t
            b = a + 1
            la = pl.multiple_of(a * CHUNK, 8)
            lb = pl.multiple_of(b * CHUNK, 8)
            stage(a, sall, si0)
            stage(a, dall, di0)
            stage(b, sall, si1)
            stage(b, dall, di1)
            ga = pltpu.async_copy(t_hbm.at[si0], ra0, sga0)
            gb = pltpu.async_copy(t_hbm.at[di0], rb0, sgb0)
            ga1c = pltpu.async_copy(t_hbm.at[si1], ra1, sga1)
            gb1c = pltpu.async_copy(t_hbm.at[di1], rb1, sgb1)
            ga.wait()
            gb.wait()
            addrows(ra0, rb0, zb0, CHUNK)
            wa = pltpu.async_copy(zb0, z_hbm.at[wid, pl.ds(la, CHUNK)], sw0)
            ga1c.wait()
            gb1c.wait()
            addrows(ra1, rb1, zb1, CHUNK)
            wb = pltpu.async_copy(zb1, z_hbm.at[wid, pl.ds(lb, CHUNK)], sw1)
            wa.wait()
            wb.wait()
            return 0
        lax.fori_loop(0, npairs, pair, 0)

        if leftover:
            j = 2 * npairs
            stage(j, sall, si0)
            stage(j, dall, di0)
            ga = pltpu.async_copy(t_hbm.at[si0], ra0, sga0)
            gb = pltpu.async_copy(t_hbm.at[di0], rb0, sgb0)
            ga.wait()
            gb.wait()
            addrows(ra0, rb0, zb0, CHUNK)
            pltpu.async_copy(zb0, z_hbm.at[wid, pl.ds(j * CHUNK, CHUNK)],
                             sw0).wait()
        if tail:
            off = base + nfull * CHUNK
            pltpu.sync_copy(src_hbm.at[pl.ds(off, tail)], sidx_t)
            pltpu.sync_copy(dst_hbm.at[pl.ds(off, tail)], didx_t)
            ga = pltpu.async_copy(t_hbm.at[sidx_t], ra_t, sga1)
            gb = pltpu.async_copy(t_hbm.at[didx_t], rb_t, sgb1)
            ga.wait()
            gb.wait()
            addrows(ra_t, rb_t, zb_t, tail)
            pltpu.async_copy(zb_t, z_hbm.at[wid, pl.ds(nfull * CHUNK, tail)],
                             sw1).wait()

    t32 = lax.bitcast_convert_type(t_packed.reshape(n, hw, 2), jnp.int32)
    z32 = k(t32, src, dst)
    return lax.bitcast_convert_type(z32, jnp.bfloat16).reshape(e, h2 // 2)


def kernel(x, edge_index, W1, b1, W2, b2, Wl1, bl1, Wl2, bl2):
    n, _ = x.shape
    h = W1.shape[1]
    src = edge_index[0]
    dst = edge_index[1]

    degp = _sc_degree(dst, n)
    y1, dinv = _matmul_scale(x, W1, degp.T, 1000)
    zeros = jnp.zeros((n, h), jnp.float32)
    sp1 = _sc_aggregate(y1, src, dst, zeros)
    y2 = _layer_mm(sp1, y1, dinv, b1, W2)
    sp2 = _sc_aggregate(y2, src, dst, zeros)
    wcat = jnp.concatenate([Wl1[:h], Wl1[h:]], axis=1)
    t_packed = _layer_mm_final(sp2, y2, dinv, b2, wcat, bl1)
    z = _sc_edge(t_packed, src, dst)
    return _final(z, Wl2, bl2)


# speedup vs baseline: 1.9371x; 1.9371x over previous
import functools

import jax
import jax.numpy as jnp
from jax import lax
from jax.experimental import pallas as pl
from jax.experimental.pallas import tpu as pltpu
from jax.experimental.pallas import tpu_sc as plsc

NW = 32
CHUNK = 128
_SC_PARAMS = pltpu.CompilerParams(needs_layout_passes=False)


def _matmul(x, w, mb):
    m, k = x.shape
    _, n = w.shape

    def body(x_ref, w_ref, o_ref):
        o_ref[...] = jnp.dot(x_ref[...], w_ref[...],
                             preferred_element_type=jnp.float32)

    return pl.pallas_call(
        body,
        grid=(m // mb,),
        in_specs=[pl.BlockSpec((mb, k), lambda i: (i, 0)),
                  pl.BlockSpec((k, n), lambda i: (0, 0))],
        out_specs=pl.BlockSpec((mb, n), lambda i: (i, 0)),
        out_shape=jax.ShapeDtypeStruct((m, n), jnp.float32),
    )(x, w)


def _matmul_scale(x, w, degp_t, mb):
    m, k = x.shape
    _, n = w.shape
    p = degp_t.shape[1]

    def body(x_ref, w_ref, d_ref, y_ref, dinv_ref):
        deg = jnp.sum(d_ref[...], axis=1) + 1.0
        dinv = lax.rsqrt(deg)
        acc = jnp.dot(x_ref[...], w_ref[...], preferred_element_type=jnp.float32)
        y_ref[...] = acc * dinv[:, None]
        dinv_ref[...] = dinv[:, None]

    return pl.pallas_call(
        body,
        grid=(m // mb,),
        in_specs=[pl.BlockSpec((mb, k), lambda i: (i, 0)),
                  pl.BlockSpec((k, n), lambda i: (0, 0)),
                  pl.BlockSpec((mb, p), lambda i: (i, 0))],
        out_specs=[pl.BlockSpec((mb, n), lambda i: (i, 0)),
                   pl.BlockSpec((mb, 1), lambda i: (i, 0))],
        out_shape=[jax.ShapeDtypeStruct((m, n), jnp.float32),
                   jax.ShapeDtypeStruct((m, 1), jnp.float32)],
    )(x, w, degp_t)


def _layer_mm(sp, y, dinv, b, w):
    _, n, h = sp.shape
    mb = 1000

    def body(sp_ref, y_ref, di_ref, b_ref, w_ref, o_ref):
        t = di_ref[...] * (sp_ref[0] + sp_ref[1] + y_ref[...]) + b_ref[...]
        hh = jnp.maximum(t, 0.0)
        o_ref[...] = jnp.dot(hh, w_ref[...],
                             preferred_element_type=jnp.float32) * di_ref[...]

    return pl.pallas_call(
        body,
        grid=(n // mb,),
        in_specs=[pl.BlockSpec((2, mb, h), lambda i: (0, i, 0)),
                  pl.BlockSpec((mb, h), lambda i: (i, 0)),
                  pl.BlockSpec((mb, 1), lambda i: (i, 0)),
                  pl.BlockSpec((h,), lambda i: (0,)),
                  pl.BlockSpec((h, h), lambda i: (0, 0))],
        out_specs=pl.BlockSpec((mb, h), lambda i: (i, 0)),
        out_shape=jax.ShapeDtypeStruct((n, h), jnp.float32),
    )(sp, y, dinv, b, w)


def _layer_mm_final(sp, y, dinv, b, wcat, bl1):
    _, n, h = sp.shape
    mb = 1000

    def body(sp_ref, y_ref, di_ref, b_ref, w_ref, bl1_ref, t_ref):
        t = di_ref[...] * (sp_ref[0] + sp_ref[1] + y_ref[...]) + b_ref[...]
        hh = jnp.maximum(t, 0.0)
        acc = jnp.dot(hh, w_ref[...], preferred_element_type=jnp.float32)
        acc = acc + jnp.concatenate(
            [bl1_ref[...], jnp.zeros_like(bl1_ref[...])])
        u = lax.bitcast_convert_type(acc, jnp.uint32)
        b16 = (u + 0x7FFF + ((u >> 16) & 1)) >> 16
        hq = h // 2
        ta = b16[:, :hq] | (b16[:, hq:h] << 16)
        tb = b16[:, h:h + hq] | (b16[:, h + hq:] << 16)
        t_ref[...] = lax.bitcast_convert_type(
            jnp.concatenate([ta, tb], axis=1), jnp.int32)

    return pl.pallas_call(
        body,
        grid=(n // mb,),
        in_specs=[pl.BlockSpec((2, mb, h), lambda i: (0, i, 0)),
                  pl.BlockSpec((mb, h), lambda i: (i, 0)),
                  pl.BlockSpec((mb, 1), lambda i: (i, 0)),
                  pl.BlockSpec((h,), lambda i: (0,)),
                  pl.BlockSpec((h, 2 * h), lambda i: (0, 0)),
                  pl.BlockSpec((h,), lambda i: (0,))],
        out_specs=pl.BlockSpec((mb, h), lambda i: (i, 0)),
        out_shape=jax.ShapeDtypeStruct((n, h), jnp.int32),
    )(sp, y, dinv, b, wcat, bl1)


def _final(z32, wl2, bl2):
    e, hw = z32.shape
    c = wl2.shape[1]
    mb = 4000

    def body(z_ref, w_ref, b_ref, o_ref):
        u = lax.bitcast_convert_type(z_ref[...], jnp.uint32)
        lo = lax.bitcast_convert_type(u << 16, jnp.float32)
        hi = lax.bitcast_convert_type(u & jnp.uint32(0xFFFF0000), jnp.float32)
        zz = jnp.maximum(jnp.concatenate([lo, hi], axis=1), 0.0)
        l = jnp.dot(zz, w_ref[...], preferred_element_type=jnp.float32) + b_ref[...]
        m = jnp.max(l, axis=1, keepdims=True)
        s = l - m
        lse = jnp.log(jnp.sum(jnp.exp(s), axis=1, keepdims=True))
        o_ref[...] = s - lse

    return pl.pallas_call(
        body,
        grid=(e // mb,),
        in_specs=[pl.BlockSpec((mb, hw), lambda i: (i, 0)),
                  pl.BlockSpec((2 * hw, c), lambda i: (0, 0)),
                  pl.BlockSpec((c,), lambda i: (0,))],
        out_specs=pl.BlockSpec((mb, c), lambda i: (i, 0)),
        out_shape=jax.ShapeDtypeStruct((e, c), jnp.float32),
    )(z32, wl2, bl2)


def _sc_degree(dst, n):
    e = dst.shape[0]
    epw = e // NW
    full = epw // 16
    rem = epw - full * 16
    mesh = plsc.VectorSubcoreMesh(core_axis_name="c", subcore_axis_name="s")

    @functools.partial(
        pl.kernel,
        out_type=jax.ShapeDtypeStruct((NW, n), jnp.float32),
        mesh=mesh,
        compiler_params=_SC_PARAMS,
        scratch_types=[
            pltpu.VMEM((epw + 16,), jnp.int32),
            pltpu.VMEM((n,), jnp.float32),
        ],
    )
    def k(dst_hbm, out_hbm, idx_v, hist_v):
        cid = lax.axis_index("c")
        sid = lax.axis_index("s")
        wid = sid * 2 + cid
        zeros16 = jnp.zeros((16,), jnp.float32)
        ones16 = jnp.ones((16,), jnp.float32)

        def zero_body(i, _):
            hist_v[pl.ds(i * 16, 16)] = zeros16
            return 0
        lax.fori_loop(0, n // 16, zero_body, 0)

        pltpu.sync_copy(dst_hbm.at[pl.ds(wid * epw, epw)], idx_v.at[pl.ds(0, epw)])

        def body(i, _):
            v = idx_v[pl.ds(i * 16, 16)]
            plsc.addupdate_scatter(hist_v, [v], ones16)
            return 0
        lax.fori_loop(0, full, body, 0)
        if rem:
            v = idx_v[pl.ds(full * 16, 16)]
            mask = lax.iota(jnp.int32, 16) < rem
            plsc.addupdate_scatter(hist_v, [v], ones16, mask=mask)

        pltpu.sync_copy(hist_v, out_hbm.at[wid])

    return k(dst)


def _sc_aggregate(y, src, dst, zeros):
    n, h = y.shape
    e = src.shape[0]
    epw = e // NW
    nfull = epw // CHUNK
    tail = epw - nfull * CHUNK
    rps = (n // (16 * 8)) * 8
    rextra = n - 16 * rps
    mesh = plsc.VectorSubcoreMesh(core_axis_name="c", subcore_axis_name="s")

    npairs = nfull // 2
    leftover = nfull - 2 * npairs

    @functools.partial(
        pl.kernel,
        out_type=jax.ShapeDtypeStruct((2, n, h), jnp.float32),
        mesh=mesh,
        compiler_params=_SC_PARAMS,
        scratch_types=[
            pltpu.VMEM((epw + 16,), jnp.int32),
            pltpu.VMEM((epw + 16,), jnp.int32),
            pltpu.VMEM((CHUNK,), jnp.int32),
            pltpu.VMEM((CHUNK,), jnp.int32),
            pltpu.VMEM((CHUNK,), jnp.int32),
            pltpu.VMEM((CHUNK,), jnp.int32),
            pltpu.VMEM((CHUNK, h), jnp.float32),
            pltpu.VMEM((CHUNK, h), jnp.float32),
            pltpu.VMEM((max(tail, 1),), jnp.int32),
            pltpu.VMEM((max(tail, 1),), jnp.int32),
            pltpu.VMEM((max(tail, 1), h), jnp.float32),
            pltpu.VMEM_SHARED((n, h), jnp.float32),
            pltpu.SemaphoreType.DMA,
            pltpu.SemaphoreType.DMA,
            pltpu.SemaphoreType.DMA,
            pltpu.SemaphoreType.DMA,
        ],
    )
    def k(y_hbm, src_hbm, dst_hbm, zero_hbm, out_hbm,
          sall, dall, si0, si1, di0, di1, rows0, rows1,
          sidx_t, didx_t, rows_t, stab,
          semg0, semg1, sems0, sems1):
        cid = lax.axis_index("c")
        sid = lax.axis_index("s")
        wid = sid * 2 + cid
        base = wid * epw
        r0 = pl.multiple_of(sid * rps, 8)
        pltpu.sync_copy(zero_hbm.at[pl.ds(r0, rps)], stab.at[pl.ds(r0, rps)])
        if rextra:
            @pl.when(sid == 15)
            def _():
                pltpu.sync_copy(zero_hbm.at[pl.ds(16 * rps, rextra)],
                                stab.at[pl.ds(16 * rps, rextra)])
        pltpu.sync_copy(src_hbm.at[pl.ds(base, epw)], sall.at[pl.ds(0, epw)])
        pltpu.sync_copy(dst_hbm.at[pl.ds(base, epw)], dall.at[pl.ds(0, epw)])
        plsc.subcore_barrier()

        def stage(j, buf_all, buf_idx, m):
            for c in range(m // 16):
                buf_idx[pl.ds(c * 16, 16)] = buf_all[pl.ds(j * CHUNK + c * 16, 16)]

        def pair(t, _):
            a = 2 * t
            b = a + 1
            stage(a, sall, si0, CHUNK)
            stage(a, dall, di0, CHUNK)
            stage(b, sall, si1, CHUNK)
            stage(b, dall, di1, CHUNK)
            ga = pltpu.async_copy(y_hbm.at[si0], rows0, semg0)
            gb = pltpu.async_copy(y_hbm.at[si1], rows1, semg1)
            ga.wait()
            sa = pltpu.async_copy(rows0, stab.at[di0], sems0, add=True)
            gb.wait()
            sb = pltpu.async_copy(rows1, stab.at[di1], sems1, add=True)
            sa.wait()
            sb.wait()
            return 0
        lax.fori_loop(0, npairs, pair, 0)

        if leftover:
            j = 2 * npairs
            stage(j, sall, si0, CHUNK)
            stage(j, dall, di0, CHUNK)
            pltpu.async_copy(y_hbm.at[si0], rows0, semg0).wait()
            pltpu.async_copy(rows0, stab.at[di0], sems0, add=True).wait()
        if tail:
            off = base + nfull * CHUNK
            pltpu.sync_copy(src_hbm.at[pl.ds(off, tail)], sidx_t)
            pltpu.sync_copy(dst_hbm.at[pl.ds(off, tail)], didx_t)
            pltpu.async_copy(y_hbm.at[sidx_t], rows_t, semg1).wait()
            pltpu.async_copy(rows_t, stab.at[didx_t], sems1, add=True).wait()

        plsc.subcore_barrier()
        pltpu.sync_copy(stab.at[pl.ds(r0, rps)],
                        out_hbm.at[cid, pl.ds(r0, rps)])
        if rextra:
            @pl.when(sid == 15)
            def _():
                pltpu.sync_copy(stab.at[pl.ds(16 * rps, rextra)],
                                out_hbm.at[cid, pl.ds(16 * rps, rextra)])

    return k(y, src, dst, zeros)


def _sc_edge(t_packed, src, dst):
    n, hw = t_packed.shape
    zw = hw // 2
    e = src.shape[0]
    epw = e // NW
    nfull = epw // CHUNK
    tail = epw - nfull * CHUNK
    mesh = plsc.VectorSubcoreMesh(core_axis_name="c", subcore_axis_name="s")

    npairs = nfull // 2
    leftover = nfull - 2 * npairs

    @functools.partial(
        pl.kernel,
        out_type=jax.ShapeDtypeStruct((NW, epw, zw), jnp.int32),
        mesh=mesh,
        compiler_params=_SC_PARAMS,
        scratch_types=[
            pltpu.VMEM((epw + 16,), jnp.int32),
            pltpu.VMEM((epw + 16,), jnp.int32),
            pltpu.VMEM((CHUNK,), jnp.int32),
            pltpu.VMEM((CHUNK,), jnp.int32),
            pltpu.VMEM((CHUNK,), jnp.int32),
            pltpu.VMEM((CHUNK,), jnp.int32),
            pltpu.VMEM((CHUNK, hw), jnp.int32),
            pltpu.VMEM((CHUNK, hw), jnp.int32),
            pltpu.VMEM((CHUNK, hw), jnp.int32),
            pltpu.VMEM((CHUNK, hw), jnp.int32),
            pltpu.VMEM((CHUNK, zw), jnp.int32),
            pltpu.VMEM((CHUNK, zw), jnp.int32),
            pltpu.VMEM((max(tail, 1),), jnp.int32),
            pltpu.VMEM((max(tail, 1),), jnp.int32),
            pltpu.VMEM((max(tail, 1), hw), jnp.int32),
            pltpu.VMEM((max(tail, 1), hw), jnp.int32),
            pltpu.VMEM((max(tail, 1), zw), jnp.int32),
            pltpu.SemaphoreType.DMA,
            pltpu.SemaphoreType.DMA,
            pltpu.SemaphoreType.DMA,
            pltpu.SemaphoreType.DMA,
            pltpu.SemaphoreType.DMA,
            pltpu.SemaphoreType.DMA,
        ],
    )
    def k(t_hbm, src_hbm, dst_hbm, z_hbm,
          sall, dall, si0, di0, si1, di1, ra0, rb0, ra1, rb1, zb0, zb1,
          sidx_t, didx_t, ra_t, rb_t, zb_t,
          sga0, sgb0, sga1, sgb1, sw0, sw1):
        cid = lax.axis_index("c")
        sid = lax.axis_index("s")
        wid = sid * 2 + cid
        base = wid * epw
        pltpu.sync_copy(src_hbm.at[pl.ds(base, epw)], sall.at[pl.ds(0, epw)])
        pltpu.sync_copy(dst_hbm.at[pl.ds(base, epw)], dall.at[pl.ds(0, epw)])

        def stage(j, buf_all, buf_idx):
            for c in range(CHUNK // 16):
                buf_idx[pl.ds(c * 16, 16)] = buf_all[pl.ds(j * CHUNK + c * 16, 16)]

        def addrows(va, vb, zb, m):
            def addrow(r, _):
                for c in range(zw // 16):
                    xa = plsc.bitcast(va[r, pl.ds(c * 16, 16)], jnp.bfloat16)
                    xb = plsc.bitcast(vb[r, pl.ds(zw + c * 16, 16)], jnp.bfloat16)
                    zb[r, pl.ds(c * 16, 16)] = plsc.bitcast(xa + xb, jnp.int32)
                return 0
            lax.fori_loop(0, m, addrow, 0)

        def pair(t, _):
            a = 2 * t
            b = a + 1
            la = pl.multiple_of(a * CHUNK, 8)
            lb = pl.multiple_of(b * CHUNK, 8)
            stage(a, sall, si0)
            stage(a, dall, di0)
            stage(b, sall, si1)
            stage(b, dall, di1)
            ga = pltpu.async_copy(t_hbm.at[si0], ra0, sga0)
            gb = pltpu.async_copy(t_hbm.at[di0], rb0, sgb0)
            ga1c = pltpu.async_copy(t_hbm.at[si1], ra1, sga1)
            gb1c = pltpu.async_copy(t_hbm.at[di1], rb1, sgb1)
            ga.wait()
            gb.wait()
            addrows(ra0, rb0, zb0, CHUNK)
            wa = pltpu.async_copy(zb0, z_hbm.at[wid, pl.ds(la, CHUNK)], sw0)
            ga1c.wait()
            gb1c.wait()
            addrows(ra1, rb1, zb1, CHUNK)
            wb = pltpu.async_copy(zb1, z_hbm.at[wid, pl.ds(lb, CHUNK)], sw1)
            wa.wait()
            wb.wait()
            return 0
        lax.fori_loop(0, npairs, pair, 0)

        if leftover:
            j = 2 * npairs
            stage(j, sall, si0)
            stage(j, dall, di0)
            ga = pltpu.async_copy(t_hbm.at[si0], ra0, sga0)
            gb = pltpu.async_copy(t_hbm.at[di0], rb0, sgb0)
            ga.wait()
            gb.wait()
            addrows(ra0, rb0, zb0, CHUNK)
            pltpu.async_copy(zb0, z_hbm.at[wid, pl.ds(j * CHUNK, CHUNK)],
                             sw0).wait()
        if tail:
            off = base + nfull * CHUNK
            pltpu.sync_copy(src_hbm.at[pl.ds(off, tail)], sidx_t)
            pltpu.sync_copy(dst_hbm.at[pl.ds(off, tail)], didx_t)
            ga = pltpu.async_copy(t_hbm.at[sidx_t], ra_t, sga1)
            gb = pltpu.async_copy(t_hbm.at[didx_t], rb_t, sgb1)
            ga.wait()
            gb.wait()
            addrows(ra_t, rb_t, zb_t, tail)
            pltpu.async_copy(zb_t, z_hbm.at[wid, pl.ds(nfull * CHUNK, tail)],
                             sw1).wait()

    return k(t_packed, src, dst).reshape(e, zw)


def kernel(x, edge_index, W1, b1, W2, b2, Wl1, bl1, Wl2, bl2):
    n, _ = x.shape
    h = W1.shape[1]
    src = edge_index[0]
    dst = edge_index[1]

    degp = _sc_degree(dst, n)
    y1, dinv = _matmul_scale(x, W1, degp.T, 1000)
    zeros = jnp.zeros((n, h), jnp.float32)
    sp1 = _sc_aggregate(y1, src, dst, zeros)
    y2 = _layer_mm(sp1, y1, dinv, b1, W2)
    sp2 = _sc_aggregate(y2, src, dst, zeros)
    wcat = jnp.concatenate([Wl1[:h], Wl1[h:]], axis=1)
    t_packed = _layer_mm_final(sp2, y2, dinv, b2, wcat, bl1)
    z = _sc_edge(t_packed, src, dst)
    return _final(z, Wl2, bl2)


# speedup vs baseline: 1.9493x; 1.0063x over previous
import functools

import jax
import jax.numpy as jnp
from jax import lax
from jax.experimental import pallas as pl
from jax.experimental.pallas import tpu as pltpu
from jax.experimental.pallas import tpu_sc as plsc

NW = 32
CHUNK = 128
_SC_PARAMS = pltpu.CompilerParams(needs_layout_passes=False)


def _matmul(x, w, mb):
    m, k = x.shape
    _, n = w.shape

    def body(x_ref, w_ref, o_ref):
        o_ref[...] = jnp.dot(x_ref[...], w_ref[...],
                             preferred_element_type=jnp.float32)

    return pl.pallas_call(
        body,
        grid=(m // mb,),
        in_specs=[pl.BlockSpec((mb, k), lambda i: (i, 0)),
                  pl.BlockSpec((k, n), lambda i: (0, 0))],
        out_specs=pl.BlockSpec((mb, n), lambda i: (i, 0)),
        out_shape=jax.ShapeDtypeStruct((m, n), jnp.float32),
    )(x, w)


def _matmul_scale(x, w, degp_t, mb):
    m, k = x.shape
    _, n = w.shape
    p = degp_t.shape[1]

    def body(x_ref, w_ref, d_ref, y_ref, dinv_ref):
        deg = jnp.sum(d_ref[...], axis=1) + 1.0
        dinv = lax.rsqrt(deg)
        acc = jnp.dot(x_ref[...], w_ref[...], preferred_element_type=jnp.float32)
        y_ref[...] = acc * dinv[:, None]
        dinv_ref[...] = dinv[:, None]

    return pl.pallas_call(
        body,
        grid=(m // mb,),
        in_specs=[pl.BlockSpec((mb, k), lambda i: (i, 0)),
                  pl.BlockSpec((k, n), lambda i: (0, 0)),
                  pl.BlockSpec((mb, p), lambda i: (i, 0))],
        out_specs=[pl.BlockSpec((mb, n), lambda i: (i, 0)),
                   pl.BlockSpec((mb, 1), lambda i: (i, 0))],
        out_shape=[jax.ShapeDtypeStruct((m, n), jnp.float32),
                   jax.ShapeDtypeStruct((m, 1), jnp.float32)],
    )(x, w, degp_t)


def _layer_mm(sp, y, dinv, b, w):
    _, n, h = sp.shape
    mb = 1000

    def body(sp_ref, y_ref, di_ref, b_ref, w_ref, o_ref):
        t = di_ref[...] * (sp_ref[0] + sp_ref[1] + y_ref[...]) + b_ref[...]
        hh = jnp.maximum(t, 0.0)
        o_ref[...] = jnp.dot(hh, w_ref[...],
                             preferred_element_type=jnp.float32) * di_ref[...]

    return pl.pallas_call(
        body,
        grid=(n // mb,),
        in_specs=[pl.BlockSpec((2, mb, h), lambda i: (0, i, 0)),
                  pl.BlockSpec((mb, h), lambda i: (i, 0)),
                  pl.BlockSpec((mb, 1), lambda i: (i, 0)),
                  pl.BlockSpec((h,), lambda i: (0,)),
                  pl.BlockSpec((h, h), lambda i: (0, 0))],
        out_specs=pl.BlockSpec((mb, h), lambda i: (i, 0)),
        out_shape=jax.ShapeDtypeStruct((n, h), jnp.float32),
    )(sp, y, dinv, b, w)


def _layer_mm_final(sp, y, dinv, b, wcat, bl1):
    _, n, h = sp.shape
    mb = 1000

    def body(sp_ref, y_ref, di_ref, b_ref, w_ref, bl1_ref, t_ref):
        t = di_ref[...] * (sp_ref[0] + sp_ref[1] + y_ref[...]) + b_ref[...]
        hh = jnp.maximum(t, 0.0)
        acc = jnp.dot(hh, w_ref[...], preferred_element_type=jnp.float32)
        acc = acc + jnp.concatenate(
            [bl1_ref[...], jnp.zeros_like(bl1_ref[...])])
        u = lax.bitcast_convert_type(acc, jnp.uint32)
        b16 = (u + 0x7FFF + ((u >> 16) & 1)) >> 16
        hq = h // 2
        ta = b16[:, :hq] | (b16[:, hq:h] << 16)
        tb = b16[:, h:h + hq] | (b16[:, h + hq:] << 16)
        t_ref[...] = lax.bitcast_convert_type(
            jnp.concatenate([ta, tb], axis=1), jnp.int32)

    return pl.pallas_call(
        body,
        grid=(n // mb,),
        in_specs=[pl.BlockSpec((2, mb, h), lambda i: (0, i, 0)),
                  pl.BlockSpec((mb, h), lambda i: (i, 0)),
                  pl.BlockSpec((mb, 1), lambda i: (i, 0)),
                  pl.BlockSpec((h,), lambda i: (0,)),
                  pl.BlockSpec((h, 2 * h), lambda i: (0, 0)),
                  pl.BlockSpec((h,), lambda i: (0,))],
        out_specs=pl.BlockSpec((mb, h), lambda i: (i, 0)),
        out_shape=jax.ShapeDtypeStruct((n, h), jnp.int32),
    )(sp, y, dinv, b, wcat, bl1)


def _final(z32, wl2, bl2):
    e, hw = z32.shape
    c = wl2.shape[1]
    mb = 4000

    def body(z_ref, w_ref, b_ref, o_ref):
        u = lax.bitcast_convert_type(z_ref[...], jnp.uint32)
        lo = lax.bitcast_convert_type(u << 16, jnp.float32)
        hi = lax.bitcast_convert_type(u & jnp.uint32(0xFFFF0000), jnp.float32)
        zz = jnp.maximum(jnp.concatenate([lo, hi], axis=1), 0.0)
        l = jnp.dot(zz, w_ref[...], preferred_element_type=jnp.float32) + b_ref[...]
        m = jnp.max(l, axis=1, keepdims=True)
        s = l - m
        lse = jnp.log(jnp.sum(jnp.exp(s), axis=1, keepdims=True))
        o_ref[...] = s - lse

    return pl.pallas_call(
        body,
        grid=(e // mb,),
        in_specs=[pl.BlockSpec((mb, hw), lambda i: (i, 0)),
                  pl.BlockSpec((2 * hw, c), lambda i: (0, 0)),
                  pl.BlockSpec((c,), lambda i: (0,))],
        out_specs=pl.BlockSpec((mb, c), lambda i: (i, 0)),
        out_shape=jax.ShapeDtypeStruct((e, c), jnp.float32),
    )(z32, wl2, bl2)


def _sc_degree(dst, n):
    e = dst.shape[0]
    epw = e // NW
    full = epw // 16
    rem = epw - full * 16
    mesh = plsc.VectorSubcoreMesh(core_axis_name="c", subcore_axis_name="s")

    @functools.partial(
        pl.kernel,
        out_type=jax.ShapeDtypeStruct((NW, n), jnp.float32),
        mesh=mesh,
        compiler_params=_SC_PARAMS,
        scratch_types=[
            pltpu.VMEM((epw + 16,), jnp.int32),
            pltpu.VMEM((n,), jnp.float32),
        ],
    )
    def k(dst_hbm, out_hbm, idx_v, hist_v):
        cid = lax.axis_index("c")
        sid = lax.axis_index("s")
        wid = sid * 2 + cid
        zeros16 = jnp.zeros((16,), jnp.float32)
        ones16 = jnp.ones((16,), jnp.float32)

        def zero_body(i, _):
            hist_v[pl.ds(i * 16, 16)] = zeros16
            return 0
        lax.fori_loop(0, n // 16, zero_body, 0)

        pltpu.sync_copy(dst_hbm.at[pl.ds(wid * epw, epw)], idx_v.at[pl.ds(0, epw)])

        def body(i, _):
            v = idx_v[pl.ds(i * 16, 16)]
            plsc.addupdate_scatter(hist_v, [v], ones16)
            return 0
        lax.fori_loop(0, full, body, 0)
        if rem:
            v = idx_v[pl.ds(full * 16, 16)]
            mask = lax.iota(jnp.int32, 16) < rem
            plsc.addupdate_scatter(hist_v, [v], ones16, mask=mask)

        pltpu.sync_copy(hist_v, out_hbm.at[wid])

    return k(dst)


def _sc_aggregate(y, src, dst, zeros):
    n, h = y.shape
    e = src.shape[0]
    epw = e // NW
    nfull = epw // CHUNK
    tail = epw - nfull * CHUNK
    rps = (n // (16 * 8)) * 8
    rextra = n - 16 * rps
    mesh = plsc.VectorSubcoreMesh(core_axis_name="c", subcore_axis_name="s")

    nd = 2
    ngroups = nfull // nd
    leftover = nfull - nd * ngroups

    @functools.partial(
        pl.kernel,
        out_type=jax.ShapeDtypeStruct((2, n, h), jnp.float32),
        mesh=mesh,
        compiler_params=_SC_PARAMS,
        scratch_types=[
            pltpu.VMEM((epw + 16,), jnp.int32),
            pltpu.VMEM((epw + 16,), jnp.int32),
        ] + [pltpu.VMEM((CHUNK,), jnp.int32)] * nd
          + [pltpu.VMEM((CHUNK,), jnp.int32)] * nd
          + [pltpu.VMEM((CHUNK, h), jnp.float32)] * nd
          + [
            pltpu.VMEM((max(tail, 1),), jnp.int32),
            pltpu.VMEM((max(tail, 1),), jnp.int32),
            pltpu.VMEM((max(tail, 1), h), jnp.float32),
            pltpu.VMEM_SHARED((n, h), jnp.float32),
        ] + [pltpu.SemaphoreType.DMA] * (2 * nd),
    )
    def k(y_hbm, src_hbm, dst_hbm, zero_hbm, out_hbm,
          sall, dall, *rest):
        si = rest[0:nd]
        di = rest[nd:2 * nd]
        rows = rest[2 * nd:3 * nd]
        sidx_t, didx_t, rows_t, stab = rest[3 * nd:3 * nd + 4]
        semg = rest[3 * nd + 4:3 * nd + 4 + nd]
        sems = rest[3 * nd + 4 + nd:3 * nd + 4 + 2 * nd]
        cid = lax.axis_index("c")
        sid = lax.axis_index("s")
        wid = sid * 2 + cid
        base = wid * epw
        r0 = pl.multiple_of(sid * rps, 8)
        pltpu.sync_copy(zero_hbm.at[pl.ds(r0, rps)], stab.at[pl.ds(r0, rps)])
        if rextra:
            @pl.when(sid == 15)
            def _():
                pltpu.sync_copy(zero_hbm.at[pl.ds(16 * rps, rextra)],
                                stab.at[pl.ds(16 * rps, rextra)])
        pltpu.sync_copy(src_hbm.at[pl.ds(base, epw)], sall.at[pl.ds(0, epw)])
        pltpu.sync_copy(dst_hbm.at[pl.ds(base, epw)], dall.at[pl.ds(0, epw)])
        plsc.subcore_barrier()

        def stage(j, buf_all, buf_idx, m):
            for c in range(m // 16):
                buf_idx[pl.ds(c * 16, 16)] = buf_all[pl.ds(j * CHUNK + c * 16, 16)]

        def group(t, _):
            j0 = nd * t
            gs = []
            for q in range(nd):
                stage(j0 + q, sall, si[q], CHUNK)
                stage(j0 + q, dall, di[q], CHUNK)
                gs.append(pltpu.async_copy(y_hbm.at[si[q]], rows[q], semg[q]))
            ss = []
            for q in range(nd):
                gs[q].wait()
                ss.append(pltpu.async_copy(rows[q], stab.at[di[q]], sems[q],
                                           add=True))
            for s in ss:
                s.wait()
            return 0
        lax.fori_loop(0, ngroups, group, 0)

        if leftover:
            j0 = nd * ngroups
            gs = []
            for q in range(leftover):
                stage(j0 + q, sall, si[q], CHUNK)
                stage(j0 + q, dall, di[q], CHUNK)
                gs.append(pltpu.async_copy(y_hbm.at[si[q]], rows[q], semg[q]))
            ss = []
            for q in range(leftover):
                gs[q].wait()
                ss.append(pltpu.async_copy(rows[q], stab.at[di[q]], sems[q],
                                           add=True))
            for s in ss:
                s.wait()
        if tail:
            off = base + nfull * CHUNK
            pltpu.sync_copy(src_hbm.at[pl.ds(off, tail)], sidx_t)
            pltpu.sync_copy(dst_hbm.at[pl.ds(off, tail)], didx_t)
            pltpu.async_copy(y_hbm.at[sidx_t], rows_t, semg[0]).wait()
            pltpu.async_copy(rows_t, stab.at[didx_t], sems[0], add=True).wait()

        plsc.subcore_barrier()
        pltpu.sync_copy(stab.at[pl.ds(r0, rps)],
                        out_hbm.at[cid, pl.ds(r0, rps)])
        if rextra:
            @pl.when(sid == 15)
            def _():
                pltpu.sync_copy(stab.at[pl.ds(16 * rps, rextra)],
                                out_hbm.at[cid, pl.ds(16 * rps, rextra)])

    return k(y, src, dst, zeros)


def _sc_edge(t_packed, src, dst):
    n, hw = t_packed.shape
    zw = hw // 2
    e = src.shape[0]
    epw = e // NW
    nfull = epw // CHUNK
    tail = epw - nfull * CHUNK
    mesh = plsc.VectorSubcoreMesh(core_axis_name="c", subcore_axis_name="s")

    npairs = nfull // 2
    leftover = nfull - 2 * npairs

    @functools.partial(
        pl.kernel,
        out_type=jax.ShapeDtypeStruct((NW, epw, zw), jnp.int32),
        mesh=mesh,
        compiler_params=_SC_PARAMS,
        scratch_types=[
            pltpu.VMEM((epw + 16,), jnp.int32),
            pltpu.VMEM((epw + 16,), jnp.int32),
            pltpu.VMEM((CHUNK,), jnp.int32),
            pltpu.VMEM((CHUNK,), jnp.int32),
            pltpu.VMEM((CHUNK,), jnp.int32),
            pltpu.VMEM((CHUNK,), jnp.int32),
            pltpu.VMEM((CHUNK, hw), jnp.int32),
            pltpu.VMEM((CHUNK, hw), jnp.int32),
            pltpu.VMEM((CHUNK, hw), jnp.int32),
            pltpu.VMEM((CHUNK, hw), jnp.int32),
            pltpu.VMEM((CHUNK, zw), jnp.int32),
            pltpu.VMEM((CHUNK, zw), jnp.int32),
            pltpu.VMEM((max(tail, 1),), jnp.int32),
            pltpu.VMEM((max(tail, 1),), jnp.int32),
            pltpu.VMEM((max(tail, 1), hw), jnp.int32),
            pltpu.VMEM((max(tail, 1), hw), jnp.int32),
            pltpu.VMEM((max(tail, 1), zw), jnp.int32),
            pltpu.SemaphoreType.DMA,
            pltpu.SemaphoreType.DMA,
            pltpu.SemaphoreType.DMA,
            pltpu.SemaphoreType.DMA,
            pltpu.SemaphoreType.DMA,
            pltpu.SemaphoreType.DMA,
        ],
    )
    def k(t_hbm, src_hbm, dst_hbm, z_hbm,
          sall, dall, si0, di0, si1, di1, ra0, rb0, ra1, rb1, zb0, zb1,
          sidx_t, didx_t, ra_t, rb_t, zb_t,
          sga0, sgb0, sga1, sgb1, sw0, sw1):
        cid = lax.axis_index("c")
        sid = lax.axis_index("s")
        wid = sid * 2 + cid
        base = wid * epw
        pltpu.sync_copy(src_hbm.at[pl.ds(base, epw)], sall.at[pl.ds(0, epw)])
        pltpu.sync_copy(dst_hbm.at[pl.ds(base, epw)], dall.at[pl.ds(0, epw)])

        def stage(j, buf_all, buf_idx):
            for c in range(CHUNK // 16):
                buf_idx[pl.ds(c * 16, 16)] = buf_all[pl.ds(j * CHUNK + c * 16, 16)]

        def addrows(va, vb, zb, m):
            def addrow(r, _):
                for c in range(zw // 16):
                    xa = plsc.bitcast(va[r, pl.ds(c * 16, 16)], jnp.bfloat16)
                    xb = plsc.bitcast(vb[r, pl.ds(zw + c * 16, 16)], jnp.bfloat16)
                    zb[r, pl.ds(c * 16, 16)] = plsc.bitcast(xa + xb, jnp.int32)
                return 0
            lax.fori_loop(0, m, addrow, 0)

        def pair(t, _):
            a = 2 * t
            b = a + 1
            la = pl.multiple_of(a * CHUNK, 8)
            lb = pl.multiple_of(b * CHUNK, 8)
            stage(a, sall, si0)
            stage(a, dall, di0)
            stage(b, sall, si1)
            stage(b, dall, di1)
            ga = pltpu.async_copy(t_hbm.at[si0], ra0, sga0)
            gb = pltpu.async_copy(t_hbm.at[di0], rb0, sgb0)
            ga1c = pltpu.async_copy(t_hbm.at[si1], ra1, sga1)
            gb1c = pltpu.async_copy(t_hbm.at[di1], rb1, sgb1)
            ga.wait()
            gb.wait()
            addrows(ra0, rb0, zb0, CHUNK)
            wa = pltpu.async_copy(zb0, z_hbm.at[wid, pl.ds(la, CHUNK)], sw0)
            ga1c.wait()
            gb1c.wait()
            addrows(ra1, rb1, zb1, CHUNK)
            wb = pltpu.async_copy(zb1, z_hbm.at[wid, pl.ds(lb, CHUNK)], sw1)
            wa.wait()
            wb.wait()
            return 0
        lax.fori_loop(0, npairs, pair, 0)

        if leftover:
            j = 2 * npairs
            stage(j, sall, si0)
            stage(j, dall, di0)
            ga = pltpu.async_copy(t_hbm.at[si0], ra0, sga0)
            gb = pltpu.async_copy(t_hbm.at[di0], rb0, sgb0)
            ga.wait()
            gb.wait()
            addrows(ra0, rb0, zb0, CHUNK)
            pltpu.async_copy(zb0, z_hbm.at[wid, pl.ds(j * CHUNK, CHUNK)],
                             sw0).wait()
        if tail:
            off = base + nfull * CHUNK
            pltpu.sync_copy(src_hbm.at[pl.ds(off, tail)], sidx_t)
            pltpu.sync_copy(dst_hbm.at[pl.ds(off, tail)], didx_t)
            ga = pltpu.async_copy(t_hbm.at[sidx_t], ra_t, sga1)
            gb = pltpu.async_copy(t_hbm.at[didx_t], rb_t, sgb1)
            ga.wait()
            gb.wait()
            addrows(ra_t, rb_t, zb_t, tail)
            pltpu.async_copy(zb_t, z_hbm.at[wid, pl.ds(nfull * CHUNK, tail)],
                             sw1).wait()

    return k(t_packed, src, dst).reshape(e, zw)


def kernel(x, edge_index, W1, b1, W2, b2, Wl1, bl1, Wl2, bl2):
    n, _ = x.shape
    h = W1.shape[1]
    src = edge_index[0]
    dst = edge_index[1]

    degp = _sc_degree(dst, n)
    y1, dinv = _matmul_scale(x, W1, degp.T, 1000)
    zeros = jnp.zeros((n, h), jnp.float32)
    sp1 = _sc_aggregate(y1, src, dst, zeros)
    y2 = _layer_mm(sp1, y1, dinv, b1, W2)
    sp2 = _sc_aggregate(y2, src, dst, zeros)
    wcat = jnp.concatenate([Wl1[:h], Wl1[h:]], axis=1)
    t_packed = _layer_mm_final(sp2, y2, dinv, b2, wcat, bl1)
    z = _sc_edge(t_packed, src, dst)
    return _final(z, Wl2, bl2)


# speedup vs baseline: 1.9589x; 1.0049x over previous
import functools

import jax
import jax.numpy as jnp
from jax import lax
from jax.experimental import pallas as pl
from jax.experimental.pallas import tpu as pltpu
from jax.experimental.pallas import tpu_sc as plsc

NW = 32
CHUNK = 128
_SC_PARAMS = pltpu.CompilerParams(needs_layout_passes=False)


def _matmul(x, w, mb):
    m, k = x.shape
    _, n = w.shape

    def body(x_ref, w_ref, o_ref):
        o_ref[...] = jnp.dot(x_ref[...], w_ref[...],
                             preferred_element_type=jnp.float32)

    return pl.pallas_call(
        body,
        grid=(m // mb,),
        in_specs=[pl.BlockSpec((mb, k), lambda i: (i, 0)),
                  pl.BlockSpec((k, n), lambda i: (0, 0))],
        out_specs=pl.BlockSpec((mb, n), lambda i: (i, 0)),
        out_shape=jax.ShapeDtypeStruct((m, n), jnp.float32),
    )(x, w)


def _matmul_scale(x, w, degp_t, mb):
    m, k = x.shape
    _, n = w.shape
    p = degp_t.shape[1]

    def body(x_ref, w_ref, d_ref, y_ref, dinv_ref):
        deg = jnp.sum(d_ref[...], axis=1) + 1.0
        dinv = lax.rsqrt(deg)
        acc = jnp.dot(x_ref[...], w_ref[...], preferred_element_type=jnp.float32)
        y_ref[...] = acc * dinv[:, None]
        dinv_ref[...] = dinv[:, None]

    return pl.pallas_call(
        body,
        grid=(m // mb,),
        in_specs=[pl.BlockSpec((mb, k), lambda i: (i, 0)),
                  pl.BlockSpec((k, n), lambda i: (0, 0)),
                  pl.BlockSpec((mb, p), lambda i: (i, 0))],
        out_specs=[pl.BlockSpec((mb, n), lambda i: (i, 0)),
                   pl.BlockSpec((mb, 1), lambda i: (i, 0))],
        out_shape=[jax.ShapeDtypeStruct((m, n), jnp.float32),
                   jax.ShapeDtypeStruct((m, 1), jnp.float32)],
    )(x, w, degp_t)


def _layer_mm(sp, y, dinv, b, w):
    _, n, h = sp.shape
    mb = 1000

    def body(sp_ref, y_ref, di_ref, b_ref, w_ref, o_ref):
        t = di_ref[...] * (sp_ref[0] + sp_ref[1] + y_ref[...]) + b_ref[...]
        hh = jnp.maximum(t, 0.0)
        o_ref[...] = jnp.dot(hh, w_ref[...],
                             preferred_element_type=jnp.float32) * di_ref[...]

    return pl.pallas_call(
        body,
        grid=(n // mb,),
        in_specs=[pl.BlockSpec((2, mb, h), lambda i: (0, i, 0)),
                  pl.BlockSpec((mb, h), lambda i: (i, 0)),
                  pl.BlockSpec((mb, 1), lambda i: (i, 0)),
                  pl.BlockSpec((h,), lambda i: (0,)),
                  pl.BlockSpec((h, h), lambda i: (0, 0))],
        out_specs=pl.BlockSpec((mb, h), lambda i: (i, 0)),
        out_shape=jax.ShapeDtypeStruct((n, h), jnp.float32),
    )(sp, y, dinv, b, w)


def _layer_mm_final(sp, y, dinv, b, wcat, bl1):
    _, n, h = sp.shape
    mb = 1000

    def body(sp_ref, y_ref, di_ref, b_ref, w_ref, bl1_ref, t_ref):
        t = di_ref[...] * (sp_ref[0] + sp_ref[1] + y_ref[...]) + b_ref[...]
        hh = jnp.maximum(t, 0.0)
        acc = jnp.dot(hh, w_ref[...], preferred_element_type=jnp.float32)
        acc = acc + jnp.concatenate(
            [bl1_ref[...], jnp.zeros_like(bl1_ref[...])])
        u = lax.bitcast_convert_type(acc, jnp.uint32)
        b16 = (u + 0x7FFF + ((u >> 16) & 1)) >> 16
        hq = h // 2
        ta = b16[:, :hq] | (b16[:, hq:h] << 16)
        tb = b16[:, h:h + hq] | (b16[:, h + hq:] << 16)
        t_ref[...] = lax.bitcast_convert_type(
            jnp.concatenate([ta, tb], axis=1), jnp.int32)

    return pl.pallas_call(
        body,
        grid=(n // mb,),
        in_specs=[pl.BlockSpec((2, mb, h), lambda i: (0, i, 0)),
                  pl.BlockSpec((mb, h), lambda i: (i, 0)),
                  pl.BlockSpec((mb, 1), lambda i: (i, 0)),
                  pl.BlockSpec((h,), lambda i: (0,)),
                  pl.BlockSpec((h, 2 * h), lambda i: (0, 0)),
                  pl.BlockSpec((h,), lambda i: (0,))],
        out_specs=pl.BlockSpec((mb, h), lambda i: (i, 0)),
        out_shape=jax.ShapeDtypeStruct((n, h), jnp.int32),
    )(sp, y, dinv, b, wcat, bl1)


def _final(z32, wl2, bl2):
    e, hw = z32.shape
    c = wl2.shape[1]
    mb = 4000

    def body(z_ref, w_ref, b_ref, o_ref):
        u = lax.bitcast_convert_type(z_ref[...], jnp.uint32)
        lo = lax.bitcast_convert_type(u << 16, jnp.float32)
        hi = lax.bitcast_convert_type(u & jnp.uint32(0xFFFF0000), jnp.float32)
        zz = jnp.maximum(jnp.concatenate([lo, hi], axis=1), 0.0)
        l = jnp.dot(zz, w_ref[...], preferred_element_type=jnp.float32) + b_ref[...]
        m = jnp.max(l, axis=1, keepdims=True)
        s = l - m
        lse = jnp.log(jnp.sum(jnp.exp(s), axis=1, keepdims=True))
        o_ref[...] = s - lse

    return pl.pallas_call(
        body,
        grid=(e // mb,),
        in_specs=[pl.BlockSpec((mb, hw), lambda i: (i, 0)),
                  pl.BlockSpec((2 * hw, c), lambda i: (0, 0)),
                  pl.BlockSpec((c,), lambda i: (0,))],
        out_specs=pl.BlockSpec((mb, c), lambda i: (i, 0)),
        out_shape=jax.ShapeDtypeStruct((e, c), jnp.float32),
    )(z32, wl2, bl2)


def _sc_degree(dst, n):
    e = dst.shape[0]
    epw = e // NW
    full = epw // 16
    rem = epw - full * 16
    mesh = plsc.VectorSubcoreMesh(core_axis_name="c", subcore_axis_name="s")

    @functools.partial(
        pl.kernel,
        out_type=jax.ShapeDtypeStruct((NW, n), jnp.float32),
        mesh=mesh,
        compiler_params=_SC_PARAMS,
        scratch_types=[
            pltpu.VMEM((epw + 16,), jnp.int32),
            pltpu.VMEM((n,), jnp.float32),
        ],
    )
    def k(dst_hbm, out_hbm, idx_v, hist_v):
        cid = lax.axis_index("c")
        sid = lax.axis_index("s")
        wid = sid * 2 + cid
        zeros16 = jnp.zeros((16,), jnp.float32)
        ones16 = jnp.ones((16,), jnp.float32)

        def zero_body(i, _):
            hist_v[pl.ds(i * 16, 16)] = zeros16
            return 0
        lax.fori_loop(0, n // 16, zero_body, 0)

        pltpu.sync_copy(dst_hbm.at[pl.ds(wid * epw, epw)], idx_v.at[pl.ds(0, epw)])

        def body(i, _):
            v = idx_v[pl.ds(i * 16, 16)]
            plsc.addupdate_scatter(hist_v, [v], ones16)
            return 0
        lax.fori_loop(0, full, body, 0)
        if rem:
            v = idx_v[pl.ds(full * 16, 16)]
            mask = lax.iota(jnp.int32, 16) < rem
            plsc.addupdate_scatter(hist_v, [v], ones16, mask=mask)

        pltpu.sync_copy(hist_v, out_hbm.at[wid])

    return k(dst)


def _sc_aggregate(y, src, dst, zeros):
    n, h = y.shape
    e = src.shape[0]
    epw = e // NW
    ACH = 96
    nfull = epw // ACH
    tail = epw - nfull * ACH
    rps = (n // (16 * 8)) * 8
    rextra = n - 16 * rps
    mesh = plsc.VectorSubcoreMesh(core_axis_name="c", subcore_axis_name="s")

    nd = 3
    ngroups = nfull // nd
    leftover = nfull - nd * ngroups

    @functools.partial(
        pl.kernel,
        out_type=jax.ShapeDtypeStruct((2, n, h), jnp.float32),
        mesh=mesh,
        compiler_params=_SC_PARAMS,
        scratch_types=[
            pltpu.VMEM((epw + 16,), jnp.int32),
            pltpu.VMEM((epw + 16,), jnp.int32),
        ] + [pltpu.VMEM((ACH,), jnp.int32)] * nd
          + [pltpu.VMEM((ACH,), jnp.int32)] * nd
          + [pltpu.VMEM((ACH, h), jnp.float32)] * nd
          + [
            pltpu.VMEM((max(tail, 1),), jnp.int32),
            pltpu.VMEM((max(tail, 1),), jnp.int32),
            pltpu.VMEM((max(tail, 1), h), jnp.float32),
            pltpu.VMEM_SHARED((n, h), jnp.float32),
        ] + [pltpu.SemaphoreType.DMA] * (2 * nd),
    )
    def k(y_hbm, src_hbm, dst_hbm, zero_hbm, out_hbm,
          sall, dall, *rest):
        si = rest[0:nd]
        di = rest[nd:2 * nd]
        rows = rest[2 * nd:3 * nd]
        sidx_t, didx_t, rows_t, stab = rest[3 * nd:3 * nd + 4]
        semg = rest[3 * nd + 4:3 * nd + 4 + nd]
        sems = rest[3 * nd + 4 + nd:3 * nd + 4 + 2 * nd]
        cid = lax.axis_index("c")
        sid = lax.axis_index("s")
        wid = sid * 2 + cid
        base = wid * epw
        r0 = pl.multiple_of(sid * rps, 8)
        pltpu.sync_copy(zero_hbm.at[pl.ds(r0, rps)], stab.at[pl.ds(r0, rps)])
        if rextra:
            @pl.when(sid == 15)
            def _():
                pltpu.sync_copy(zero_hbm.at[pl.ds(16 * rps, rextra)],
                                stab.at[pl.ds(16 * rps, rextra)])
        pltpu.sync_copy(src_hbm.at[pl.ds(base, epw)], sall.at[pl.ds(0, epw)])
        pltpu.sync_copy(dst_hbm.at[pl.ds(base, epw)], dall.at[pl.ds(0, epw)])
        plsc.subcore_barrier()

        def stage(j, buf_all, buf_idx, m):
            for c in range(m // 16):
                buf_idx[pl.ds(c * 16, 16)] = buf_all[pl.ds(j * ACH + c * 16, 16)]

        def group(t, _):
            j0 = nd * t
            gs = []
            for q in range(nd):
                stage(j0 + q, sall, si[q], ACH)
                stage(j0 + q, dall, di[q], ACH)
                gs.append(pltpu.async_copy(y_hbm.at[si[q]], rows[q], semg[q]))
            ss = []
            for q in range(nd):
                gs[q].wait()
                ss.append(pltpu.async_copy(rows[q], stab.at[di[q]], sems[q],
                                           add=True))
            for s in ss:
                s.wait()
            return 0
        lax.fori_loop(0, ngroups, group, 0)

        if leftover:
            j0 = nd * ngroups
            gs = []
            for q in range(leftover):
                stage(j0 + q, sall, si[q], ACH)
                stage(j0 + q, dall, di[q], ACH)
                gs.append(pltpu.async_copy(y_hbm.at[si[q]], rows[q], semg[q]))
            ss = []
            for q in range(leftover):
                gs[q].wait()
                ss.append(pltpu.async_copy(rows[q], stab.at[di[q]], sems[q],
                                           add=True))
            for s in ss:
                s.wait()
        if tail:
            off = base + nfull * ACH
            pltpu.sync_copy(src_hbm.at[pl.ds(off, tail)], sidx_t)
            pltpu.sync_copy(dst_hbm.at[pl.ds(off, tail)], didx_t)
            pltpu.async_copy(y_hbm.at[sidx_t], rows_t, semg[0]).wait()
            pltpu.async_copy(rows_t, stab.at[didx_t], sems[0], add=True).wait()

        plsc.subcore_barrier()
        pltpu.sync_copy(stab.at[pl.ds(r0, rps)],
                        out_hbm.at[cid, pl.ds(r0, rps)])
        if rextra:
            @pl.when(sid == 15)
            def _():
                pltpu.sync_copy(stab.at[pl.ds(16 * rps, rextra)],
                                out_hbm.at[cid, pl.ds(16 * rps, rextra)])

    return k(y, src, dst, zeros)


def _sc_edge(t_packed, src, dst):
    n, hw = t_packed.shape
    zw = hw // 2
    e = src.shape[0]
    epw = e // NW
    nfull = epw // CHUNK
    tail = epw - nfull * CHUNK
    mesh = plsc.VectorSubcoreMesh(core_axis_name="c", subcore_axis_name="s")

    npairs = nfull // 2
    leftover = nfull - 2 * npairs

    @functools.partial(
        pl.kernel,
        out_type=jax.ShapeDtypeStruct((NW, epw, zw), jnp.int32),
        mesh=mesh,
        compiler_params=_SC_PARAMS,
        scratch_types=[
            pltpu.VMEM((epw + 16,), jnp.int32),
            pltpu.VMEM((epw + 16,), jnp.int32),
            pltpu.VMEM((CHUNK,), jnp.int32),
            pltpu.VMEM((CHUNK,), jnp.int32),
            pltpu.VMEM((CHUNK,), jnp.int32),
            pltpu.VMEM((CHUNK,), jnp.int32),
            pltpu.VMEM((CHUNK, hw), jnp.int32),
            pltpu.VMEM((CHUNK, hw), jnp.int32),
            pltpu.VMEM((CHUNK, hw), jnp.int32),
            pltpu.VMEM((CHUNK, hw), jnp.int32),
            pltpu.VMEM((CHUNK, zw), jnp.int32),
            pltpu.VMEM((CHUNK, zw), jnp.int32),
            pltpu.VMEM((max(tail, 1),), jnp.int32),
            pltpu.VMEM((max(tail, 1),), jnp.int32),
            pltpu.VMEM((max(tail, 1), hw), jnp.int32),
            pltpu.VMEM((max(tail, 1), hw), jnp.int32),
            pltpu.VMEM((max(tail, 1), zw), jnp.int32),
            pltpu.SemaphoreType.DMA,
            pltpu.SemaphoreType.DMA,
            pltpu.SemaphoreType.DMA,
            pltpu.SemaphoreType.DMA,
            pltpu.SemaphoreType.DMA,
            pltpu.SemaphoreType.DMA,
        ],
    )
    def k(t_hbm, src_hbm, dst_hbm, z_hbm,
          sall, dall, si0, di0, si1, di1, ra0, rb0, ra1, rb1, zb0, zb1,
          sidx_t, didx_t, ra_t, rb_t, zb_t,
          sga0, sgb0, sga1, sgb1, sw0, sw1):
        cid = lax.axis_index("c")
        sid = lax.axis_index("s")
        wid = sid * 2 + cid
        base = wid * epw
        pltpu.sync_copy(src_hbm.at[pl.ds(base, epw)], sall.at[pl.ds(0, epw)])
        pltpu.sync_copy(dst_hbm.at[pl.ds(base, epw)], dall.at[pl.ds(0, epw)])

        def stage(j, buf_all, buf_idx):
            for c in range(CHUNK // 16):
                buf_idx[pl.ds(c * 16, 16)] = buf_all[pl.ds(j * CHUNK + c * 16, 16)]

        def addrows(va, vb, zb, m):
            def addrow(r, _):
                for c in range(zw // 16):
                    xa = plsc.bitcast(va[r, pl.ds(c * 16, 16)], jnp.bfloat16)
                    xb = plsc.bitcast(vb[r, pl.ds(zw + c * 16, 16)], jnp.bfloat16)
                    zb[r, pl.ds(c * 16, 16)] = plsc.bitcast(xa + xb, jnp.int32)
                return 0
            lax.fori_loop(0, m, addrow, 0)

        def pair(t, _):
            a = 2 * t
            b = a + 1
            la = pl.multiple_of(a * CHUNK, 8)
            lb = pl.multiple_of(b * CHUNK, 8)
            stage(a, sall, si0)
            stage(a, dall, di0)
            stage(b, sall, si1)
            stage(b, dall, di1)
            ga = pltpu.async_copy(t_hbm.at[si0], ra0, sga0)
            gb = pltpu.async_copy(t_hbm.at[di0], rb0, sgb0)
            ga1c = pltpu.async_copy(t_hbm.at[si1], ra1, sga1)
            gb1c = pltpu.async_copy(t_hbm.at[di1], rb1, sgb1)
            ga.wait()
            gb.wait()
            addrows(ra0, rb0, zb0, CHUNK)
            wa = pltpu.async_copy(zb0, z_hbm.at[wid, pl.ds(la, CHUNK)], sw0)
            ga1c.wait()
            gb1c.wait()
            addrows(ra1, rb1, zb1, CHUNK)
            wb = pltpu.async_copy(zb1, z_hbm.at[wid, pl.ds(lb, CHUNK)], sw1)
            wa.wait()
            wb.wait()
            return 0
        lax.fori_loop(0, npairs, pair, 0)

        if leftover:
            j = 2 * npairs
            stage(j, sall, si0)
            stage(j, dall, di0)
            ga = pltpu.async_copy(t_hbm.at[si0], ra0, sga0)
            gb = pltpu.async_copy(t_hbm.at[di0], rb0, sgb0)
            ga.wait()
            gb.wait()
            addrows(ra0, rb0, zb0, CHUNK)
            pltpu.async_copy(zb0, z_hbm.at[wid, pl.ds(j * CHUNK, CHUNK)],
                             sw0).wait()
        if tail:
            off = base + nfull * CHUNK
            pltpu.sync_copy(src_hbm.at[pl.ds(off, tail)], sidx_t)
            pltpu.sync_copy(dst_hbm.at[pl.ds(off, tail)], didx_t)
            ga = pltpu.async_copy(t_hbm.at[sidx_t], ra_t, sga1)
            gb = pltpu.async_copy(t_hbm.at[didx_t], rb_t, sgb1)
            ga.wait()
            gb.wait()
            addrows(ra_t, rb_t, zb_t, tail)
            pltpu.async_copy(zb_t, z_hbm.at[wid, pl.ds(nfull * CHUNK, tail)],
                             sw1).wait()

    return k(t_packed, src, dst).reshape(e, zw)


def kernel(x, edge_index, W1, b1, W2, b2, Wl1, bl1, Wl2, bl2):
    n, _ = x.shape
    h = W1.shape[1]
    src = edge_index[0]
    dst = edge_index[1]

    degp = _sc_degree(dst, n)
    y1, dinv = _matmul_scale(x, W1, degp.T, 1000)
    zeros = jnp.zeros((n, h), jnp.float32)
    sp1 = _sc_aggregate(y1, src, dst, zeros)
    y2 = _layer_mm(sp1, y1, dinv, b1, W2)
    sp2 = _sc_aggregate(y2, src, dst, zeros)
    wcat = jnp.concatenate([Wl1[:h], Wl1[h:]], axis=1)
    t_packed = _layer_mm_final(sp2, y2, dinv, b2, wcat, bl1)
    z = _sc_edge(t_packed, src, dst)
    return _final(z, Wl2, bl2)
